# baseline (device time: 30043 ns/iter reference)
import jax
import jax.numpy as jnp
from jax import lax
from jax.experimental import pallas as pl
from jax.experimental.pallas import tpu as pltpu

N_DEV = 4
B, SQ, DM = 2, 128, 512
HL, DH = 4, 64
SKV_SH = 128
WIN = 128
BF16 = jnp.bfloat16


def kernel(x, Wq, K_ext, V_ext, Wo):
    xb = x.astype(BF16)
    wqb = Wq.astype(BF16)
    wob = Wo.astype(BF16)
    kb = jnp.transpose(K_ext.astype(BF16), (2, 0, 1, 3))
    vb = jnp.transpose(V_ext.astype(BF16), (2, 0, 1, 3))

    def body(x_ref, wq_ref, k_ref, v_ref, wo_ref, out_ref,
             kv_mine, out_parts, kvr, kvs, o_send, o_recv):
        my = lax.axis_index("i")
        left = lax.rem(my + N_DEV - 1, N_DEV)
        right = lax.rem(my + 1, N_DEV)
        diag = lax.rem(my + 2, N_DEV)

        def copy(src, dst, ssem, rsem, tgt):
            return pltpu.make_async_remote_copy(
                src_ref=src, dst_ref=dst, send_sem=ssem, recv_sem=rsem,
                device_id=(tgt,), device_id_type=pl.DeviceIdType.MESH)

        def hgrp(ref, g):
            return ref.at[pl.ds(HL * g, HL)]

        barrier = pltpu.get_barrier_semaphore()
        for nbr in (left, right, diag):
            pl.semaphore_signal(barrier, inc=1, device_id=(nbr,),
                                device_id_type=pl.DeviceIdType.MESH)
        pl.semaphore_wait(barrier, 3)

        def kv_sends(me, chunk):
            peers = ((me + 2) % N_DEV, (me + 3) % N_DEV, (me + 1) % N_DEV)
            for i, t in enumerate(peers):
                copy(hgrp(k_ref, t), kv_mine.at[chunk, 0],
                     kvs.at[i], kvr.at[chunk, 0], t).start()
            for i, t in enumerate(peers):
                copy(hgrp(v_ref, t), kv_mine.at[chunk, 1],
                     kvs.at[3 + i], kvr.at[chunk, 1], t).start()

        @pl.when(my == 0)
        def _():
            kv_mine[0, 0] = k_ref[pl.ds(0, HL)]
            kv_mine[0, 1] = v_ref[pl.ds(0, HL)]
            kv_sends(0, 0)

        @pl.when(my == 1)
        def _():
            kv_mine[1, 0] = k_ref[pl.ds(HL, HL)]
            kv_mine[1, 1] = v_ref[pl.ds(HL, HL)]
            kv_sends(1, 1)

        q = jnp.dot(x_ref[...].reshape(B * SQ, DM), wq_ref[...],
                    preferred_element_type=jnp.float32)
        qg = (q.astype(BF16).reshape(B, SQ, HL, DH)
              .transpose(2, 0, 1, 3).reshape(HL * B, SQ, DH))

        def wait_part(c, kv):
            copy(kv_mine.at[c, kv], kv_mine.at[c, kv], kvs.at[7],
                 kvr.at[c, kv], 0).wait_recv()

        def wait_mine(kv):
            @pl.when(my == 0)
            def _():
                wait_part(1, kv)

            @pl.when(my == 1)
            def _():
                wait_part(0, kv)

            @pl.when(my >= 2)
            def _():
                wait_part(0, kv)
                wait_part(1, kv)

        wait_mine(0)

        k01 = jnp.concatenate([kv_mine[0, 0], kv_mine[1, 0]],
                              axis=2).reshape(HL * B, 2 * SKV_SH, DH)

        scores = jnp.einsum('gsd,gtd->gst', qg, k01,
                            preferred_element_type=jnp.float32) * 0.125
        si = lax.broadcasted_iota(jnp.int32, (SQ, 2 * SKV_SH), 0)
        ti = lax.broadcasted_iota(jnp.int32, (SQ, 2 * SKV_SH), 1)
        mask = (ti - si) <= WIN
        scores = jnp.where(mask[None], scores, -1e9)

        m = jnp.max(scores, axis=-1, keepdims=True)
        e = jnp.exp(scores - m)
        w = (e / jnp.sum(e, axis=-1, keepdims=True)).astype(BF16)

        wait_mine(1)
        v01 = jnp.concatenate([kv_mine[0, 1], kv_mine[1, 1]],
                              axis=2).reshape(HL * B, 2 * SKV_SH, DH)

        ctx = jnp.einsum('gst,gtd->gsd', w, v01,
                         preferred_element_type=jnp.float32)
        ctx2 = (ctx.astype(BF16).reshape(HL, B, SQ, DH)
                .transpose(1, 2, 0, 3).reshape(B * SQ, HL * DH))
        part = jnp.dot(ctx2, wo_ref[...], preferred_element_type=jnp.float32)
        out_parts[0] = part.astype(BF16).reshape(B, SQ, DM)

        s_r = copy(out_parts.at[0], out_parts.at[1], o_send.at[0],
                   o_recv.at[0], right)
        s_l = copy(out_parts.at[0], out_parts.at[2], o_send.at[1],
                   o_recv.at[1], left)
        s_d = copy(out_parts.at[0], out_parts.at[3], o_send.at[2],
                   o_recv.at[2], diag)
        s_d.start()
        s_r.start()
        s_l.start()
        s_r.wait_recv()
        s_l.wait_recv()
        s_d.wait_recv()

        out_ref[...] = ((out_parts[0].astype(jnp.float32)
                         + out_parts[1].astype(jnp.float32))
                        + (out_parts[2].astype(jnp.float32)
                           + out_parts[3].astype(jnp.float32)))

        s_r.wait_send()
        s_l.wait_send()
        s_d.wait_send()

        @pl.when(my <= 1)
        def _():
            for i in range(6):
                copy(kv_mine.at[0, 0], kv_mine.at[0, 0], kvs.at[i],
                     kvr.at[0, 0], 0).wait_send()

    return pl.pallas_call(
        body,
        out_shape=jax.ShapeDtypeStruct((B, SQ, DM), jnp.float32),
        in_specs=[pl.BlockSpec(memory_space=pltpu.VMEM)] * 5,
        out_specs=pl.BlockSpec(memory_space=pltpu.VMEM),
        scratch_shapes=[
            pltpu.VMEM((2, 2, HL, B, SKV_SH, DH), BF16),
            pltpu.VMEM((N_DEV, B, SQ, DM), BF16),
            pltpu.SemaphoreType.DMA((2, 2)),
            pltpu.SemaphoreType.DMA((8,)),
            pltpu.SemaphoreType.DMA((3,)),
            pltpu.SemaphoreType.DMA((3,)),
        ],
        compiler_params=pltpu.CompilerParams(collective_id=0),
    )(xb, wqb, kb, vb, wob)
